# Initial kernel scaffold; baseline (speedup 1.0000x reference)
import jax
import jax.numpy as jnp
from jax.experimental import pallas as pl


def kernel(x, edge_index, c, W1, b1, gamma1, beta1, W2, b2, eps):
    raise NotImplementedError("write your pallas kernel here")


# speedup vs baseline: 5.9735x; 5.9735x over previous
import functools

import jax
import jax.numpy as jnp
from jax import lax
from jax.experimental import pallas as pl
from jax.experimental.pallas import tpu as pltpu
from jax.experimental.pallas import tpu_sc as plsc

N = 10000
E = 320000
IN = 128
CD = 16
S = 4
EMB = 128
HID = 2 * EMB
DIN = IN + CD
F = IN + S * CD
HW = F // 2

NC = 2
NS = 16

CH = 128
EPAD = 327680
CHUNKS = EPAD // CH
TCH = CHUNKS // NS
TPAIRS = TCH // 2
NPAD = 10240
RPT = NPAD // NS

BN = 2000
GRID = N // BN


def _sc_scatter(feat_a, feat_b, src2d, dst2d, zrows):
    mesh = plsc.VectorSubcoreMesh(core_axis_name="c", subcore_axis_name="s")

    @functools.partial(
        pl.kernel,
        out_type=jax.ShapeDtypeStruct((NC, NPAD, HW), jnp.float32),
        mesh=mesh,
        scratch_types=[
            pltpu.VMEM((TCH, CH), jnp.int32),
            pltpu.VMEM((TCH, CH), jnp.int32),
            pltpu.VMEM((CH, HW), jnp.float32),
            pltpu.VMEM((CH, HW), jnp.float32),
            pltpu.VMEM_SHARED((NPAD, HW), jnp.float32),
            pltpu.SemaphoreType.DMA,
            pltpu.SemaphoreType.DMA,
        ],
        compiler_params=pltpu.CompilerParams(use_tc_tiling_on_sc=False),
    )
    def k(fa_hbm, fb_hbm, src_hbm, dst_hbm, z_hbm, out_hbm,
          src_v, dst_v, g0, g1, acc, sem0, sem1):
        cid = lax.axis_index("c")
        sid = lax.axis_index("s")
        pltpu.sync_copy(z_hbm, acc.at[pl.ds(sid * RPT, RPT)])
        pltpu.sync_copy(src_hbm.at[pl.ds(sid * TCH, TCH)], src_v)
        pltpu.sync_copy(dst_hbm.at[pl.ds(sid * TCH, TCH)], dst_v)
        plsc.subcore_barrier()

        def chunk_loop(f_hbm):
            def body(jj, carry):
                j0 = 2 * jj
                j1 = j0 + 1
                d0 = pltpu.async_copy(f_hbm.at[src_v.at[j0]], g0, sem0)
                d1 = pltpu.async_copy(f_hbm.at[src_v.at[j1]], g1, sem1)
                d0.wait()
                pltpu.sync_copy(g0, acc.at[dst_v.at[j0]], add=True)
                d1.wait()
                pltpu.sync_copy(g1, acc.at[dst_v.at[j1]], add=True)
                return carry
            lax.fori_loop(0, TPAIRS, body, 0)

        @pl.when(cid == 0)
        def _():
            chunk_loop(fa_hbm)

        @pl.when(cid == 1)
        def _():
            chunk_loop(fb_hbm)

        plsc.subcore_barrier()
        pltpu.sync_copy(acc.at[pl.ds(sid * RPT, RPT)],
                        out_hbm.at[cid, pl.ds(sid * RPT, RPT)])

    return k(feat_a, feat_b, src2d, dst2d, zrows)


def _stats_body(feat_ref, p0_ref, p1_ref, eps_ref, w1x_ref, w1c_ref, b1_ref,
                out_ref):
    i = pl.program_id(0)
    e1 = 1.0 + eps_ref[0, 0]
    agg = jnp.concatenate([p0_ref[...], p1_ref[...]], axis=1)
    z = e1 * feat_ref[...] + agg
    base = jnp.dot(z[:, :IN], w1x_ref[...], preferred_element_type=jnp.float32)
    tot = jnp.zeros((HID,), jnp.float32)
    totq = jnp.zeros((HID,), jnp.float32)
    for s in range(S):
        lo = IN + s * CD
        a = base + jnp.dot(z[:, lo:lo + CD], w1c_ref[...],
                           preferred_element_type=jnp.float32) + b1_ref[0, :]
        tot = tot + a.sum(axis=0)
        totq = totq + (a * a).sum(axis=0)

    @pl.when(i == 0)
    def _():
        out_ref[...] = jnp.zeros_like(out_ref)

    out_ref[0:1, :] += tot[None, :]
    out_ref[1:2, :] += totq[None, :]


def _mlp_body(feat_ref, p0_ref, p1_ref, eps_ref, stats_ref, w1x_ref, w1c_ref,
              b1_ref, gamma_ref, beta_ref, w2_ref, b2_ref, out_ref):
    e1 = 1.0 + eps_ref[0, 0]
    m = float(N * S)
    mean = stats_ref[0, :] / m
    var = stats_ref[1, :] / m - mean * mean
    scale = gamma_ref[0, :] * lax.rsqrt(var + 1e-5)
    shift = beta_ref[0, :] - mean * scale
    agg = jnp.concatenate([p0_ref[...], p1_ref[...]], axis=1)
    z = e1 * feat_ref[...] + agg
    base = jnp.dot(z[:, :IN], w1x_ref[...], preferred_element_type=jnp.float32)
    w2 = w2_ref[...]
    for s in range(S):
        lo = IN + s * CD
        a = base + jnp.dot(z[:, lo:lo + CD], w1c_ref[...],
                           preferred_element_type=jnp.float32) + b1_ref[0, :]
        h = jnp.maximum(a * scale + shift, 0.0)
        o = jnp.dot(h, w2, preferred_element_type=jnp.float32) + b2_ref[0, :]
        out_ref[:, s * EMB:(s + 1) * EMB] = o


def kernel(x, edge_index, c, W1, b1, gamma1, beta1, W2, b2, eps):
    feat = jnp.concatenate([x, c.reshape(N, S * CD)], axis=1)
    zrow8 = jnp.zeros((8, HW), jnp.float32)
    feat_a = jnp.concatenate([feat[:, :HW], zrow8], axis=0)
    feat_b = jnp.concatenate([feat[:, HW:], zrow8], axis=0)
    pad = EPAD - E
    src2d = jnp.concatenate(
        [edge_index[0], jnp.full((pad,), N, jnp.int32)]).reshape(-1, CH)
    dst2d = jnp.concatenate(
        [edge_index[1], jnp.zeros((pad,), jnp.int32)]).reshape(-1, CH)
    zrows = jnp.zeros((RPT, HW), jnp.float32)

    partials = _sc_scatter(feat_a, feat_b, src2d, dst2d, zrows)
    p0 = partials[0, :N]
    p1 = partials[1, :N]

    eps2d = eps.reshape(1, 1)
    w1x = W1[:IN, :]
    w1c = W1[IN:, :]
    b1r = b1.reshape(1, HID)
    gammar = gamma1.reshape(1, HID)
    betar = beta1.reshape(1, HID)
    b2r = b2.reshape(1, EMB)

    feat_spec = pl.BlockSpec((BN, F), lambda i: (i, 0))
    half_spec = pl.BlockSpec((BN, HW), lambda i: (i, 0))
    full = lambda shape: pl.BlockSpec(shape, lambda i: tuple(0 for _ in shape))

    stats = pl.pallas_call(
        _stats_body,
        grid=(GRID,),
        in_specs=[
            feat_spec, half_spec, half_spec,
            full((1, 1)), full((IN, HID)), full((CD, HID)), full((1, HID)),
        ],
        out_specs=full((8, HID)),
        out_shape=jax.ShapeDtypeStruct((8, HID), jnp.float32),
    )(feat, p0, p1, eps2d, w1x, w1c, b1r)

    out = pl.pallas_call(
        _mlp_body,
        grid=(GRID,),
        in_specs=[
            feat_spec, half_spec, half_spec,
            full((1, 1)), full((8, HID)), full((IN, HID)), full((CD, HID)),
            full((1, HID)), full((1, HID)), full((1, HID)),
            full((HID, EMB)), full((1, EMB)),
        ],
        out_specs=pl.BlockSpec((BN, S * EMB), lambda i: (i, 0)),
        out_shape=jax.ShapeDtypeStruct((N, S * EMB), jnp.float32),
    )(feat, p0, p1, eps2d, stats, w1x, w1c, b1r, gammar, betar, W2, b2r)

    return out


# speedup vs baseline: 9.2011x; 1.5403x over previous
import functools

import jax
import jax.numpy as jnp
from jax import lax
from jax.experimental import pallas as pl
from jax.experimental.pallas import tpu as pltpu
from jax.experimental.pallas import tpu_sc as plsc

N = 10000
E = 320000
IN = 128
CD = 16
S = 4
EMB = 128
HID = 2 * EMB
DIN = IN + CD
F = IN + S * CD
HW = F // 2

NC = 2
NS = 16

CH = 40
DEPTH = 8
EPAD = 322560
TCH = EPAD // CH // NS
NPAD = 10112
RPT = NPAD // NS

BN = 2000
GRID = N // BN


def _sc_scatter(feat_a, feat_b, src2d, dst2d, zrows):
    mesh = plsc.VectorSubcoreMesh(core_axis_name="c", subcore_axis_name="s")

    idx_scratch = [pltpu.VMEM((CH,), jnp.int32) for _ in range(2 * DEPTH)]
    idx_sems = [pltpu.SemaphoreType.DMA for _ in range(2 * DEPTH)]

    @functools.partial(
        pl.kernel,
        out_type=jax.ShapeDtypeStruct((NC, NPAD, HW), jnp.float32),
        mesh=mesh,
        scratch_types=[
            pltpu.VMEM((CH, HW), jnp.float32),
            pltpu.VMEM((CH, HW), jnp.float32),
            pltpu.VMEM_SHARED((NPAD, HW), jnp.float32),
            pltpu.VMEM_SHARED((NPAD, HW), jnp.float32),
            pltpu.SemaphoreType.DMA,
            pltpu.SemaphoreType.DMA,
        ] + idx_scratch + idx_sems,
        compiler_params=pltpu.CompilerParams(use_tc_tiling_on_sc=False),
    )
    def k(fa_hbm, fb_hbm, src_hbm, dst_hbm, z_hbm, out_hbm,
          g0, g1, table, acc, semg0, semg1, *idx_args):
        sbuf = idx_args[0:DEPTH]
        dbuf = idx_args[DEPTH:2 * DEPTH]
        ssem = idx_args[2 * DEPTH:3 * DEPTH]
        dsem = idx_args[3 * DEPTH:4 * DEPTH]
        gbuf = (g0, g1)
        gsem = (semg0, semg1)
        cid = lax.axis_index("c")
        sid = lax.axis_index("s")
        rows = pl.ds(sid * RPT, RPT)
        base = sid * TCH

        @pl.when(cid == 0)
        def _():
            pltpu.sync_copy(fa_hbm.at[rows], table.at[rows])

        @pl.when(cid == 1)
        def _():
            pltpu.sync_copy(fb_hbm.at[rows], table.at[rows])

        pltpu.sync_copy(z_hbm, acc.at[rows])

        for s_ in range(DEPTH):
            pltpu.async_copy(src_hbm.at[base + s_], sbuf[s_], ssem[s_])
            pltpu.async_copy(dst_hbm.at[base + s_], dbuf[s_], dsem[s_])
        plsc.subcore_barrier()

        def body(g_, carry):
            for k_ in range(DEPTH):
                chunk = g_ * DEPTH + k_
                gs = k_ % 2
                pltpu.make_async_copy(src_hbm.at[base], sbuf[k_],
                                      ssem[k_]).wait()
                pltpu.make_async_copy(dst_hbm.at[base], dbuf[k_],
                                      dsem[k_]).wait()
                pltpu.async_copy(table.at[sbuf[k_]], gbuf[gs],
                                 gsem[gs]).wait()
                pltpu.sync_copy(gbuf[gs], acc.at[dbuf[k_]], add=True)
                @pl.when(chunk + DEPTH < TCH)
                def _():
                    pltpu.async_copy(src_hbm.at[base + chunk + DEPTH],
                                     sbuf[k_], ssem[k_])
                    pltpu.async_copy(dst_hbm.at[base + chunk + DEPTH],
                                     dbuf[k_], dsem[k_])
            return carry

        lax.fori_loop(0, TCH // DEPTH, body, 0)
        plsc.subcore_barrier()
        pltpu.sync_copy(acc.at[rows], out_hbm.at[cid, rows])

    return k(feat_a, feat_b, src2d, dst2d, zrows)


def _stats_body(feat_ref, p0_ref, p1_ref, eps_ref, w1x_ref, w1c_ref, b1_ref,
                out_ref):
    i = pl.program_id(0)
    e1 = 1.0 + eps_ref[0, 0]
    agg = jnp.concatenate([p0_ref[...], p1_ref[...]], axis=1)
    z = e1 * feat_ref[...] + agg
    base = jnp.dot(z[:, :IN], w1x_ref[...], preferred_element_type=jnp.float32)
    tot = jnp.zeros((HID,), jnp.float32)
    totq = jnp.zeros((HID,), jnp.float32)
    for s in range(S):
        lo = IN + s * CD
        a = base + jnp.dot(z[:, lo:lo + CD], w1c_ref[...],
                           preferred_element_type=jnp.float32) + b1_ref[0, :]
        tot = tot + a.sum(axis=0)
        totq = totq + (a * a).sum(axis=0)

    @pl.when(i == 0)
    def _():
        out_ref[...] = jnp.zeros_like(out_ref)

    out_ref[0:1, :] += tot[None, :]
    out_ref[1:2, :] += totq[None, :]


def _mlp_body(feat_ref, p0_ref, p1_ref, eps_ref, stats_ref, w1x_ref, w1c_ref,
              b1_ref, gamma_ref, beta_ref, w2_ref, b2_ref, out_ref):
    e1 = 1.0 + eps_ref[0, 0]
    m = float(N * S)
    mean = stats_ref[0, :] / m
    var = stats_ref[1, :] / m - mean * mean
    scale = gamma_ref[0, :] * lax.rsqrt(var + 1e-5)
    shift = beta_ref[0, :] - mean * scale
    agg = jnp.concatenate([p0_ref[...], p1_ref[...]], axis=1)
    z = e1 * feat_ref[...] + agg
    base = jnp.dot(z[:, :IN], w1x_ref[...], preferred_element_type=jnp.float32)
    w2 = w2_ref[...]
    for s in range(S):
        lo = IN + s * CD
        a = base + jnp.dot(z[:, lo:lo + CD], w1c_ref[...],
                           preferred_element_type=jnp.float32) + b1_ref[0, :]
        h = jnp.maximum(a * scale + shift, 0.0)
        o = jnp.dot(h, w2, preferred_element_type=jnp.float32) + b2_ref[0, :]
        out_ref[:, s * EMB:(s + 1) * EMB] = o


def kernel(x, edge_index, c, W1, b1, gamma1, beta1, W2, b2, eps):
    feat = jnp.concatenate([x, c.reshape(N, S * CD)], axis=1)
    zrowp = jnp.zeros((NPAD - N, HW), jnp.float32)
    feat_a = jnp.concatenate([feat[:, :HW], zrowp], axis=0)
    feat_b = jnp.concatenate([feat[:, HW:], zrowp], axis=0)
    pad = EPAD - E
    src2d = jnp.concatenate(
        [edge_index[0], jnp.full((pad,), N, jnp.int32)]).reshape(-1, CH)
    dst2d = jnp.concatenate(
        [edge_index[1], jnp.zeros((pad,), jnp.int32)]).reshape(-1, CH)
    zrows = jnp.zeros((RPT, HW), jnp.float32)

    partials = _sc_scatter(feat_a, feat_b, src2d, dst2d, zrows)
    p0 = partials[0, :N]
    p1 = partials[1, :N]

    eps2d = eps.reshape(1, 1)
    w1x = W1[:IN, :]
    w1c = W1[IN:, :]
    b1r = b1.reshape(1, HID)
    gammar = gamma1.reshape(1, HID)
    betar = beta1.reshape(1, HID)
    b2r = b2.reshape(1, EMB)

    feat_spec = pl.BlockSpec((BN, F), lambda i: (i, 0))
    half_spec = pl.BlockSpec((BN, HW), lambda i: (i, 0))
    full = lambda shape: pl.BlockSpec(shape, lambda i: tuple(0 for _ in shape))

    stats = pl.pallas_call(
        _stats_body,
        grid=(GRID,),
        in_specs=[
            feat_spec, half_spec, half_spec,
            full((1, 1)), full((IN, HID)), full((CD, HID)), full((1, HID)),
        ],
        out_specs=full((8, HID)),
        out_shape=jax.ShapeDtypeStruct((8, HID), jnp.float32),
    )(feat, p0, p1, eps2d, w1x, w1c, b1r)

    out = pl.pallas_call(
        _mlp_body,
        grid=(GRID,),
        in_specs=[
            feat_spec, half_spec, half_spec,
            full((1, 1)), full((8, HID)), full((IN, HID)), full((CD, HID)),
            full((1, HID)), full((1, HID)), full((1, HID)),
            full((HID, EMB)), full((1, EMB)),
        ],
        out_specs=pl.BlockSpec((BN, S * EMB), lambda i: (i, 0)),
        out_shape=jax.ShapeDtypeStruct((N, S * EMB), jnp.float32),
    )(feat, p0, p1, eps2d, stats, w1x, w1c, b1r, gammar, betar, W2, b2r)

    return out


# speedup vs baseline: 11.7693x; 1.2791x over previous
import functools

import jax
import jax.numpy as jnp
from jax import lax
from jax.experimental import pallas as pl
from jax.experimental.pallas import tpu as pltpu
from jax.experimental.pallas import tpu_sc as plsc

N = 10000
E = 320000
IN = 128
CD = 16
S = 4
EMB = 128
HID = 2 * EMB
DIN = IN + CD
F = IN + S * CD
HW = F // 2

NC = 2
NS = 16

CH = 40
DEPTH = 8
PF = 6
EPAD = 322560
TCH = EPAD // CH // NS
NPAD = 10112
RPT = NPAD // NS

BN = 2000
GRID = N // BN


def _sc_scatter(feat_a, feat_b, src2d, dst2d, zrows):
    mesh = plsc.VectorSubcoreMesh(core_axis_name="c", subcore_axis_name="s")

    idx_scratch = [pltpu.VMEM((CH,), jnp.int32) for _ in range(2 * DEPTH)]
    idx_sems = [pltpu.SemaphoreType.DMA for _ in range(2 * DEPTH)]

    @functools.partial(
        pl.kernel,
        out_type=jax.ShapeDtypeStruct((NC, NPAD, HW), jnp.float32),
        mesh=mesh,
        scratch_types=[
            pltpu.VMEM((CH, HW), jnp.float32),
            pltpu.VMEM((CH, HW), jnp.float32),
            pltpu.VMEM_SHARED((NPAD, HW), jnp.float32),
            pltpu.VMEM_SHARED((NPAD, HW), jnp.float32),
            pltpu.SemaphoreType.DMA,
            pltpu.SemaphoreType.DMA,
            pltpu.SemaphoreType.DMA,
            pltpu.SemaphoreType.DMA,
        ] + idx_scratch + idx_sems,
        compiler_params=pltpu.CompilerParams(use_tc_tiling_on_sc=False),
    )
    def k(fa_hbm, fb_hbm, src_hbm, dst_hbm, z_hbm, out_hbm,
          g0, g1, table, acc, semg0, semg1, sems0, sems1, *idx_args):
        sbuf = idx_args[0:DEPTH]
        dbuf = idx_args[DEPTH:2 * DEPTH]
        ssem = idx_args[2 * DEPTH:3 * DEPTH]
        dsem = idx_args[3 * DEPTH:4 * DEPTH]
        gbuf = (g0, g1)
        gsem = (semg0, semg1)
        ssem2 = (sems0, sems1)
        cid = lax.axis_index("c")
        sid = lax.axis_index("s")
        rows = pl.ds(sid * RPT, RPT)
        base = sid * TCH

        @pl.when(cid == 0)
        def _():
            pltpu.sync_copy(fa_hbm.at[rows], table.at[rows])

        @pl.when(cid == 1)
        def _():
            pltpu.sync_copy(fb_hbm.at[rows], table.at[rows])

        pltpu.sync_copy(z_hbm, acc.at[rows])

        for s_ in range(PF):
            pltpu.async_copy(src_hbm.at[base + s_], sbuf[s_], ssem[s_])
            pltpu.async_copy(dst_hbm.at[base + s_], dbuf[s_], dsem[s_])
        plsc.subcore_barrier()

        def body(g_, carry):
            for k_ in range(DEPTH):
                chunk = g_ * DEPTH + k_
                gs = k_ % 2
                pltpu.make_async_copy(src_hbm.at[base], sbuf[k_],
                                      ssem[k_]).wait()
                pltpu.make_async_copy(dst_hbm.at[base], dbuf[k_],
                                      dsem[k_]).wait()
                @pl.when(chunk >= 2)
                def _():
                    pltpu.make_async_copy(gbuf[gs], acc.at[dbuf[k_]],
                                          ssem2[gs]).wait()
                pltpu.async_copy(table.at[sbuf[k_]], gbuf[gs],
                                 gsem[gs]).wait()
                pltpu.async_copy(gbuf[gs], acc.at[dbuf[k_]], ssem2[gs],
                                 add=True)
                pf = (k_ + PF) % DEPTH
                @pl.when(chunk + PF < TCH)
                def _():
                    pltpu.async_copy(src_hbm.at[base + chunk + PF],
                                     sbuf[pf], ssem[pf])
                    pltpu.async_copy(dst_hbm.at[base + chunk + PF],
                                     dbuf[pf], dsem[pf])
            return carry

        lax.fori_loop(0, TCH // DEPTH, body, 0)
        pltpu.make_async_copy(gbuf[0], acc.at[dbuf[0]], ssem2[0]).wait()
        pltpu.make_async_copy(gbuf[1], acc.at[dbuf[1]], ssem2[1]).wait()
        plsc.subcore_barrier()
        pltpu.sync_copy(acc.at[rows], out_hbm.at[cid, rows])

    return k(feat_a, feat_b, src2d, dst2d, zrows)


def _stats_body(feat_ref, p0_ref, p1_ref, eps_ref, w1x_ref, w1c_ref, b1_ref,
                out_ref):
    i = pl.program_id(0)
    e1 = 1.0 + eps_ref[0, 0]
    agg = jnp.concatenate([p0_ref[...], p1_ref[...]], axis=1)
    z = e1 * feat_ref[...] + agg
    base = jnp.dot(z[:, :IN], w1x_ref[...], preferred_element_type=jnp.float32)
    tot = jnp.zeros((HID,), jnp.float32)
    totq = jnp.zeros((HID,), jnp.float32)
    for s in range(S):
        lo = IN + s * CD
        a = base + jnp.dot(z[:, lo:lo + CD], w1c_ref[...],
                           preferred_element_type=jnp.float32) + b1_ref[0, :]
        tot = tot + a.sum(axis=0)
        totq = totq + (a * a).sum(axis=0)

    @pl.when(i == 0)
    def _():
        out_ref[...] = jnp.zeros_like(out_ref)

    out_ref[0:1, :] += tot[None, :]
    out_ref[1:2, :] += totq[None, :]


def _mlp_body(feat_ref, p0_ref, p1_ref, eps_ref, stats_ref, w1x_ref, w1c_ref,
              b1_ref, gamma_ref, beta_ref, w2_ref, b2_ref, out_ref):
    e1 = 1.0 + eps_ref[0, 0]
    m = float(N * S)
    mean = stats_ref[0, :] / m
    var = stats_ref[1, :] / m - mean * mean
    scale = gamma_ref[0, :] * lax.rsqrt(var + 1e-5)
    shift = beta_ref[0, :] - mean * scale
    agg = jnp.concatenate([p0_ref[...], p1_ref[...]], axis=1)
    z = e1 * feat_ref[...] + agg
    base = jnp.dot(z[:, :IN], w1x_ref[...], preferred_element_type=jnp.float32)
    w2 = w2_ref[...]
    for s in range(S):
        lo = IN + s * CD
        a = base + jnp.dot(z[:, lo:lo + CD], w1c_ref[...],
                           preferred_element_type=jnp.float32) + b1_ref[0, :]
        h = jnp.maximum(a * scale + shift, 0.0)
        o = jnp.dot(h, w2, preferred_element_type=jnp.float32) + b2_ref[0, :]
        out_ref[:, s * EMB:(s + 1) * EMB] = o


def kernel(x, edge_index, c, W1, b1, gamma1, beta1, W2, b2, eps):
    feat = jnp.concatenate([x, c.reshape(N, S * CD)], axis=1)
    zrowp = jnp.zeros((NPAD - N, HW), jnp.float32)
    feat_a = jnp.concatenate([feat[:, :HW], zrowp], axis=0)
    feat_b = jnp.concatenate([feat[:, HW:], zrowp], axis=0)
    pad = EPAD - E
    src2d = jnp.concatenate(
        [edge_index[0], jnp.full((pad,), N, jnp.int32)]).reshape(-1, CH)
    dst2d = jnp.concatenate(
        [edge_index[1], jnp.zeros((pad,), jnp.int32)]).reshape(-1, CH)
    zrows = jnp.zeros((RPT, HW), jnp.float32)

    partials = _sc_scatter(feat_a, feat_b, src2d, dst2d, zrows)
    p0 = partials[0, :N]
    p1 = partials[1, :N]

    eps2d = eps.reshape(1, 1)
    w1x = W1[:IN, :]
    w1c = W1[IN:, :]
    b1r = b1.reshape(1, HID)
    gammar = gamma1.reshape(1, HID)
    betar = beta1.reshape(1, HID)
    b2r = b2.reshape(1, EMB)

    feat_spec = pl.BlockSpec((BN, F), lambda i: (i, 0))
    half_spec = pl.BlockSpec((BN, HW), lambda i: (i, 0))
    full = lambda shape: pl.BlockSpec(shape, lambda i: tuple(0 for _ in shape))

    stats = pl.pallas_call(
        _stats_body,
        grid=(GRID,),
        in_specs=[
            feat_spec, half_spec, half_spec,
            full((1, 1)), full((IN, HID)), full((CD, HID)), full((1, HID)),
        ],
        out_specs=full((8, HID)),
        out_shape=jax.ShapeDtypeStruct((8, HID), jnp.float32),
    )(feat, p0, p1, eps2d, w1x, w1c, b1r)

    out = pl.pallas_call(
        _mlp_body,
        grid=(GRID,),
        in_specs=[
            feat_spec, half_spec, half_spec,
            full((1, 1)), full((8, HID)), full((IN, HID)), full((CD, HID)),
            full((1, HID)), full((1, HID)), full((1, HID)),
            full((HID, EMB)), full((1, EMB)),
        ],
        out_specs=pl.BlockSpec((BN, S * EMB), lambda i: (i, 0)),
        out_shape=jax.ShapeDtypeStruct((N, S * EMB), jnp.float32),
    )(feat, p0, p1, eps2d, stats, w1x, w1c, b1r, gammar, betar, W2, b2r)

    return out


# speedup vs baseline: 12.4647x; 1.0591x over previous
import functools

import jax
import jax.numpy as jnp
from jax import lax
from jax.experimental import pallas as pl
from jax.experimental.pallas import tpu as pltpu
from jax.experimental.pallas import tpu_sc as plsc

N = 10000
E = 320000
IN = 128
CD = 16
S = 4
EMB = 128
HID = 2 * EMB
DIN = IN + CD
F = IN + S * CD
HW = F // 2

NC = 2
NS = 16

CH = 40
DEPTH = 8
PF = 6
EPAD = 322560
TCH = EPAD // CH // NS
NPAD = 10112
RPT = NPAD // NS

BN = 2000
GRID = N // BN


def _sc_scatter(feat_a, feat_b, src2d, dst2d, zrows):
    mesh = plsc.VectorSubcoreMesh(core_axis_name="c", subcore_axis_name="s")

    idx_scratch = [pltpu.VMEM((CH,), jnp.int32) for _ in range(2 * DEPTH)]
    idx_sems = [pltpu.SemaphoreType.DMA for _ in range(2 * DEPTH)]

    @functools.partial(
        pl.kernel,
        out_type=jax.ShapeDtypeStruct((NC, NPAD, HW), jnp.float32),
        mesh=mesh,
        scratch_types=[
            pltpu.VMEM((CH, HW), jnp.float32),
            pltpu.VMEM((CH, HW), jnp.float32),
            pltpu.VMEM_SHARED((NPAD, HW), jnp.float32),
            pltpu.VMEM_SHARED((NPAD, HW), jnp.float32),
            pltpu.SemaphoreType.DMA,
            pltpu.SemaphoreType.DMA,
            pltpu.SemaphoreType.DMA,
            pltpu.SemaphoreType.DMA,
        ] + idx_scratch + idx_sems,
        compiler_params=pltpu.CompilerParams(use_tc_tiling_on_sc=False),
    )
    def k(fa_hbm, fb_hbm, src_hbm, dst_hbm, z_hbm, out_hbm,
          g0, g1, table, acc, semg0, semg1, sems0, sems1, *idx_args):
        sbuf = idx_args[0:DEPTH]
        dbuf = idx_args[DEPTH:2 * DEPTH]
        ssem = idx_args[2 * DEPTH:3 * DEPTH]
        dsem = idx_args[3 * DEPTH:4 * DEPTH]
        gbuf = (g0, g1)
        gsem = (semg0, semg1)
        ssem2 = (sems0, sems1)
        cid = lax.axis_index("c")
        sid = lax.axis_index("s")
        rows = pl.ds(sid * RPT, RPT)
        base = sid * TCH

        @pl.when(cid == 0)
        def _():
            pltpu.sync_copy(fa_hbm.at[rows], table.at[rows])

        @pl.when(cid == 1)
        def _():
            pltpu.sync_copy(fb_hbm.at[rows], table.at[rows])

        pltpu.sync_copy(z_hbm, acc.at[rows])

        for s_ in range(PF):
            pltpu.async_copy(src_hbm.at[base + s_], sbuf[s_], ssem[s_])
            pltpu.async_copy(dst_hbm.at[base + s_], dbuf[s_], dsem[s_])
        plsc.subcore_barrier()

        def body(g_, carry):
            for k_ in range(DEPTH):
                chunk = g_ * DEPTH + k_
                gs = k_ % 2
                pltpu.make_async_copy(src_hbm.at[base], sbuf[k_],
                                      ssem[k_]).wait()
                pltpu.make_async_copy(dst_hbm.at[base], dbuf[k_],
                                      dsem[k_]).wait()
                @pl.when(chunk >= 2)
                def _():
                    pltpu.make_async_copy(gbuf[gs], acc.at[dbuf[k_]],
                                          ssem2[gs]).wait()
                pltpu.async_copy(table.at[sbuf[k_]], gbuf[gs],
                                 gsem[gs]).wait()
                pltpu.async_copy(gbuf[gs], acc.at[dbuf[k_]], ssem2[gs],
                                 add=True)
                pf = (k_ + PF) % DEPTH
                @pl.when(chunk + PF < TCH)
                def _():
                    pltpu.async_copy(src_hbm.at[base + chunk + PF],
                                     sbuf[pf], ssem[pf])
                    pltpu.async_copy(dst_hbm.at[base + chunk + PF],
                                     dbuf[pf], dsem[pf])
            return carry

        lax.fori_loop(0, TCH // DEPTH, body, 0)
        pltpu.make_async_copy(gbuf[0], acc.at[dbuf[0]], ssem2[0]).wait()
        pltpu.make_async_copy(gbuf[1], acc.at[dbuf[1]], ssem2[1]).wait()
        plsc.subcore_barrier()
        pltpu.sync_copy(acc.at[rows], out_hbm.at[cid, rows])

    return k(feat_a, feat_b, src2d, dst2d, zrows)


def _stats_body(x_ref, c_ref, p0_ref, p1_ref, eps_ref, w1x_ref, w1c_ref,
                b1_ref, out_ref):
    i = pl.program_id(0)
    e1 = 1.0 + eps_ref[0, 0]
    agg = jnp.concatenate([p0_ref[0], p1_ref[0]], axis=1)
    feat = jnp.concatenate([x_ref[...], c_ref[...]], axis=1)
    z = e1 * feat + agg
    base = jnp.dot(z[:, :IN], w1x_ref[...], preferred_element_type=jnp.float32)
    tot = jnp.zeros((HID,), jnp.float32)
    totq = jnp.zeros((HID,), jnp.float32)
    for s in range(S):
        lo = IN + s * CD
        a = base + jnp.dot(z[:, lo:lo + CD], w1c_ref[...],
                           preferred_element_type=jnp.float32) + b1_ref[0, :]
        tot = tot + a.sum(axis=0)
        totq = totq + (a * a).sum(axis=0)

    @pl.when(i == 0)
    def _():
        out_ref[...] = jnp.zeros_like(out_ref)

    out_ref[0:1, :] += tot[None, :]
    out_ref[1:2, :] += totq[None, :]


def _mlp_body(x_ref, c_ref, p0_ref, p1_ref, eps_ref, stats_ref, w1x_ref,
              w1c_ref, b1_ref, gamma_ref, beta_ref, w2_ref, b2_ref, out_ref):
    e1 = 1.0 + eps_ref[0, 0]
    m = float(N * S)
    mean = stats_ref[0, :] / m
    var = stats_ref[1, :] / m - mean * mean
    scale = gamma_ref[0, :] * lax.rsqrt(var + 1e-5)
    shift = beta_ref[0, :] - mean * scale
    agg = jnp.concatenate([p0_ref[0], p1_ref[0]], axis=1)
    feat = jnp.concatenate([x_ref[...], c_ref[...]], axis=1)
    z = e1 * feat + agg
    base = jnp.dot(z[:, :IN], w1x_ref[...], preferred_element_type=jnp.float32)
    w2 = w2_ref[...]
    for s in range(S):
        lo = IN + s * CD
        a = base + jnp.dot(z[:, lo:lo + CD], w1c_ref[...],
                           preferred_element_type=jnp.float32) + b1_ref[0, :]
        h = jnp.maximum(a * scale + shift, 0.0)
        o = jnp.dot(h, w2, preferred_element_type=jnp.float32) + b2_ref[0, :]
        out_ref[:, s * EMB:(s + 1) * EMB] = o


def kernel(x, edge_index, c, W1, b1, gamma1, beta1, W2, b2, eps):
    c2d_ = c.reshape(N, S * CD)
    zrowp = jnp.zeros((NPAD - N, HW), jnp.float32)
    feat_a = jnp.concatenate([x[:, :HW], zrowp], axis=0)
    feat_b = jnp.concatenate(
        [jnp.concatenate([x[:, HW:], c2d_], axis=1), zrowp], axis=0)
    pad = EPAD - E
    src2d = jnp.concatenate(
        [edge_index[0], jnp.full((pad,), N, jnp.int32)]).reshape(-1, CH)
    dst2d = jnp.concatenate(
        [edge_index[1], jnp.zeros((pad,), jnp.int32)]).reshape(-1, CH)
    zrows = jnp.zeros((RPT, HW), jnp.float32)

    partials = _sc_scatter(feat_a, feat_b, src2d, dst2d, zrows)
    c2d = c2d_

    eps2d = eps.reshape(1, 1)
    w1x = W1[:IN, :]
    w1c = W1[IN:, :]
    b1r = b1.reshape(1, HID)
    gammar = gamma1.reshape(1, HID)
    betar = beta1.reshape(1, HID)
    b2r = b2.reshape(1, EMB)

    x_spec = pl.BlockSpec((BN, IN), lambda i: (i, 0))
    c_spec = pl.BlockSpec((BN, S * CD), lambda i: (i, 0))
    p0_spec = pl.BlockSpec((1, BN, HW), lambda i: (0, i, 0))
    p1_spec = pl.BlockSpec((1, BN, HW), lambda i: (1, i, 0))
    full = lambda shape: pl.BlockSpec(shape, lambda i: tuple(0 for _ in shape))

    stats = pl.pallas_call(
        _stats_body,
        grid=(GRID,),
        in_specs=[
            x_spec, c_spec, p0_spec, p1_spec,
            full((1, 1)), full((IN, HID)), full((CD, HID)), full((1, HID)),
        ],
        out_specs=full((8, HID)),
        out_shape=jax.ShapeDtypeStruct((8, HID), jnp.float32),
    )(x, c2d, partials, partials, eps2d, w1x, w1c, b1r)

    out = pl.pallas_call(
        _mlp_body,
        grid=(GRID,),
        in_specs=[
            x_spec, c_spec, p0_spec, p1_spec,
            full((1, 1)), full((8, HID)), full((IN, HID)), full((CD, HID)),
            full((1, HID)), full((1, HID)), full((1, HID)),
            full((HID, EMB)), full((1, EMB)),
        ],
        out_specs=pl.BlockSpec((BN, S * EMB), lambda i: (i, 0)),
        out_shape=jax.ShapeDtypeStruct((N, S * EMB), jnp.float32),
    )(x, c2d, partials, partials, eps2d, stats, w1x, w1c, b1r, gammar,
      betar, W2, b2r)

    return out


# speedup vs baseline: 12.6192x; 1.0124x over previous
import functools

import jax
import jax.numpy as jnp
from jax import lax
from jax.experimental import pallas as pl
from jax.experimental.pallas import tpu as pltpu
from jax.experimental.pallas import tpu_sc as plsc

N = 10000
E = 320000
IN = 128
CD = 16
S = 4
EMB = 128
HID = 2 * EMB
DIN = IN + CD
F = IN + S * CD
HW = F // 2

NC = 2
NS = 16

CH = 40
DEPTH = 8
PF = 6
EPAD = 322560
TCH = EPAD // CH // NS
NPAD = 10112
RPT = NPAD // NS

BN = 2000
GRID = N // BN


def _sc_scatter(feat_a, feat_b, src2d, dst2d, zrows):
    mesh = plsc.VectorSubcoreMesh(core_axis_name="c", subcore_axis_name="s")

    idx_scratch = [pltpu.VMEM((CH,), jnp.int32) for _ in range(2 * DEPTH)]
    idx_sems = [pltpu.SemaphoreType.DMA for _ in range(2 * DEPTH)]

    @functools.partial(
        pl.kernel,
        out_type=jax.ShapeDtypeStruct((NC, NPAD, HW), jnp.float32),
        mesh=mesh,
        scratch_types=[
            pltpu.VMEM((CH, HW), jnp.float32),
            pltpu.VMEM((CH, HW), jnp.float32),
            pltpu.VMEM_SHARED((NPAD, HW), jnp.float32),
            pltpu.VMEM_SHARED((NPAD, HW), jnp.float32),
            pltpu.SemaphoreType.DMA,
            pltpu.SemaphoreType.DMA,
            pltpu.SemaphoreType.DMA,
            pltpu.SemaphoreType.DMA,
        ] + idx_scratch + idx_sems,
        compiler_params=pltpu.CompilerParams(use_tc_tiling_on_sc=False),
    )
    def k(fa_hbm, fb_hbm, src_hbm, dst_hbm, z_hbm, out_hbm,
          g0, g1, table, acc, semg0, semg1, sems0, sems1, *idx_args):
        sbuf = idx_args[0:DEPTH]
        dbuf = idx_args[DEPTH:2 * DEPTH]
        ssem = idx_args[2 * DEPTH:3 * DEPTH]
        dsem = idx_args[3 * DEPTH:4 * DEPTH]
        gbuf = (g0, g1)
        gsem = (semg0, semg1)
        ssem2 = (sems0, sems1)
        cid = lax.axis_index("c")
        sid = lax.axis_index("s")
        rows = pl.ds(sid * RPT, RPT)
        base = sid * TCH

        @pl.when(cid == 0)
        def _():
            pltpu.sync_copy(fa_hbm.at[rows], table.at[rows])

        @pl.when(cid == 1)
        def _():
            pltpu.sync_copy(fb_hbm.at[rows], table.at[rows])

        pltpu.sync_copy(z_hbm, acc.at[rows])

        for s_ in range(PF):
            pltpu.async_copy(src_hbm.at[base + s_], sbuf[s_], ssem[s_])
            pltpu.async_copy(dst_hbm.at[base + s_], dbuf[s_], dsem[s_])
        plsc.subcore_barrier()

        def body(g_, carry):
            for k_ in range(DEPTH):
                chunk = g_ * DEPTH + k_
                gs = k_ % 2
                pltpu.make_async_copy(src_hbm.at[base], sbuf[k_],
                                      ssem[k_]).wait()
                pltpu.make_async_copy(dst_hbm.at[base], dbuf[k_],
                                      dsem[k_]).wait()
                @pl.when(chunk >= 2)
                def _():
                    pltpu.make_async_copy(gbuf[gs], acc.at[dbuf[k_]],
                                          ssem2[gs]).wait()
                pltpu.async_copy(table.at[sbuf[k_]], gbuf[gs],
                                 gsem[gs]).wait()
                pltpu.async_copy(gbuf[gs], acc.at[dbuf[k_]], ssem2[gs],
                                 add=True)
                pf = (k_ + PF) % DEPTH
                @pl.when(chunk + PF < TCH)
                def _():
                    pltpu.async_copy(src_hbm.at[base + chunk + PF],
                                     sbuf[pf], ssem[pf])
                    pltpu.async_copy(dst_hbm.at[base + chunk + PF],
                                     dbuf[pf], dsem[pf])
            return carry

        lax.fori_loop(0, TCH // DEPTH, body, 0)
        pltpu.make_async_copy(gbuf[0], acc.at[dbuf[0]], ssem2[0]).wait()
        pltpu.make_async_copy(gbuf[1], acc.at[dbuf[1]], ssem2[1]).wait()
        plsc.subcore_barrier()
        pltpu.sync_copy(acc.at[rows], out_hbm.at[cid, rows])

    return k(feat_a, feat_b, src2d, dst2d, zrows)


def _mlp_body(x_ref, c_ref, p0_ref, p1_ref, eps_ref, w1x_ref, w1c_ref,
              b1_ref, gamma_ref, beta_ref, w2_ref, b2_ref, out_ref,
              z_ref, st_ref):
    p = pl.program_id(0)
    i = pl.program_id(1)
    e1 = 1.0 + eps_ref[0, 0]

    @pl.when(p == 0)
    def _():
        agg = jnp.concatenate([p0_ref[0], p1_ref[0]], axis=1)
        feat = jnp.concatenate([x_ref[...], c_ref[...]], axis=1)
        z = e1 * feat + agg
        z_ref[pl.ds(i * BN, BN), :] = z
        base = jnp.dot(z[:, :IN], w1x_ref[...],
                       preferred_element_type=jnp.float32)
        tot = jnp.zeros((HID,), jnp.float32)
        totq = jnp.zeros((HID,), jnp.float32)
        for s in range(S):
            lo = IN + s * CD
            a = base + jnp.dot(z[:, lo:lo + CD], w1c_ref[...],
                               preferred_element_type=jnp.float32) \
                + b1_ref[0, :]
            tot = tot + a.sum(axis=0)
            totq = totq + (a * a).sum(axis=0)

        @pl.when(i == 0)
        def _():
            st_ref[...] = jnp.zeros_like(st_ref)

        st_ref[0:1, :] += tot[None, :]
        st_ref[1:2, :] += totq[None, :]

    @pl.when(p == 1)
    def _():
        m = float(N * S)
        mean = st_ref[0, :] / m
        var = st_ref[1, :] / m - mean * mean
        scale = gamma_ref[0, :] * lax.rsqrt(var + 1e-5)
        shift = beta_ref[0, :] - mean * scale
        z = z_ref[pl.ds(i * BN, BN), :]
        base = jnp.dot(z[:, :IN], w1x_ref[...],
                       preferred_element_type=jnp.float32)
        w2 = w2_ref[...]
        for s in range(S):
            lo = IN + s * CD
            a = base + jnp.dot(z[:, lo:lo + CD], w1c_ref[...],
                               preferred_element_type=jnp.float32) \
                + b1_ref[0, :]
            h = jnp.maximum(a * scale + shift, 0.0)
            o = jnp.dot(h, w2, preferred_element_type=jnp.float32) \
                + b2_ref[0, :]
            out_ref[:, s * EMB:(s + 1) * EMB] = o


def kernel(x, edge_index, c, W1, b1, gamma1, beta1, W2, b2, eps):
    c2d_ = c.reshape(N, S * CD)
    zrowp = jnp.zeros((NPAD - N, HW), jnp.float32)
    feat_a = jnp.concatenate([x[:, :HW], zrowp], axis=0)
    feat_b = jnp.concatenate(
        [jnp.concatenate([x[:, HW:], c2d_], axis=1), zrowp], axis=0)
    pad = EPAD - E
    src2d = jnp.concatenate(
        [edge_index[0], jnp.full((pad,), N, jnp.int32)]).reshape(-1, CH)
    dst2d = jnp.concatenate(
        [edge_index[1], jnp.zeros((pad,), jnp.int32)]).reshape(-1, CH)
    zrows = jnp.zeros((RPT, HW), jnp.float32)

    partials = _sc_scatter(feat_a, feat_b, src2d, dst2d, zrows)
    c2d = c2d_

    eps2d = eps.reshape(1, 1)
    w1x = W1[:IN, :]
    w1c = W1[IN:, :]
    b1r = b1.reshape(1, HID)
    gammar = gamma1.reshape(1, HID)
    betar = beta1.reshape(1, HID)
    b2r = b2.reshape(1, EMB)

    blk = lambda p_, i_: (i_ * (1 - p_), 0)
    x_spec = pl.BlockSpec((BN, IN), blk)
    c_spec = pl.BlockSpec((BN, S * CD), blk)
    p0_spec = pl.BlockSpec((1, BN, HW), lambda p_, i_: (0, i_ * (1 - p_), 0))
    p1_spec = pl.BlockSpec((1, BN, HW), lambda p_, i_: (1, i_ * (1 - p_), 0))
    full = lambda shape: pl.BlockSpec(
        shape, lambda p_, i_: tuple(0 for _ in shape))

    out = pl.pallas_call(
        _mlp_body,
        grid=(2, GRID),
        in_specs=[
            x_spec, c_spec, p0_spec, p1_spec,
            full((1, 1)), full((IN, HID)), full((CD, HID)),
            full((1, HID)), full((1, HID)), full((1, HID)),
            full((HID, EMB)), full((1, EMB)),
        ],
        out_specs=pl.BlockSpec((BN, S * EMB),
                               lambda p_, i_: (i_ * p_, 0)),
        out_shape=jax.ShapeDtypeStruct((N, S * EMB), jnp.float32),
        scratch_shapes=[
            pltpu.VMEM((N, F), jnp.float32),
            pltpu.VMEM((8, HID), jnp.float32),
        ],
    )(x, c2d, partials, partials, eps2d, w1x, w1c, b1r, gammar, betar,
      W2, b2r)

    return out
